# Initial kernel scaffold; baseline (speedup 1.0000x reference)
import jax
import jax.numpy as jnp
from jax.experimental import pallas as pl


def kernel(tensor, params):
    raise NotImplementedError("write your pallas kernel here")


# speedup vs baseline: 68.0349x; 68.0349x over previous
import functools
import math

import numpy as np
import jax
import jax.numpy as jnp
from jax.experimental import pallas as pl
from jax.experimental.pallas import tpu as pltpu

_H = 4
_KD = 16
_VD = 16
_FACTOR = 5
_LAYERS = 3
_DCONV = 2
_D = 128


def _rotl(x, r):
    return ((x << np.uint32(r)) | (x >> np.uint32(32 - r))).astype(np.uint32)


def _threefry2x32(k0, k1, x0, x1):
    with np.errstate(over='ignore'):
        ks0, ks1 = np.uint32(k0), np.uint32(k1)
        ks2 = np.uint32(ks0 ^ ks1 ^ np.uint32(0x1BD11BDA))
        ks = [ks0, ks1, ks2]
        x0 = (x0 + ks0).astype(np.uint32)
        x1 = (x1 + ks1).astype(np.uint32)
        rot = [[13, 15, 26, 6], [17, 29, 16, 24]]
        for i in range(5):
            for r in rot[i % 2]:
                x0 = (x0 + x1).astype(np.uint32)
                x1 = _rotl(x1, r) ^ x0
            x0 = (x0 + ks[(i + 1) % 3]).astype(np.uint32)
            x1 = (x1 + ks[(i + 2) % 3] + np.uint32(i + 1)).astype(np.uint32)
    return x0, x1


def _np_bits(k0, k1, n):
    i = np.arange(n, dtype=np.uint64)
    hi = (i >> np.uint64(32)).astype(np.uint32)
    lo = (i & np.uint64(0xFFFFFFFF)).astype(np.uint32)
    y0, y1 = _threefry2x32(k0, k1, hi, lo)
    return y0 ^ y1


def _np_randint(k0, k1, t, u):
    hi0, hi1 = _threefry2x32(k0, k1, np.uint32(0), np.uint32(0))
    lo0, lo1 = _threefry2x32(k0, k1, np.uint32(0), np.uint32(1))
    y = _np_bits(hi0, hi1, t * u)
    z = _np_bits(lo0, lo1, t * u)
    span = np.uint32(t)
    mult = np.uint32((int(65536 % span) ** 2) % int(span))
    val = ((y % span) * mult + (z % span)) % span
    return val.reshape(t, u).astype(np.int32)


@functools.lru_cache(maxsize=None)
def _sample_count_t(layer: int, t: int, u_part: int):
    k0, k1 = _threefry2x32(np.uint32(0), np.uint32(42),
                           np.uint32(0), np.uint32(layer))
    idx = _np_randint(k0, k1, t, u_part)
    c = np.zeros((t, t), np.int8)
    np.add.at(c, (np.repeat(np.arange(t), u_part), idx.ravel()), 1)
    return np.ascontiguousarray(c.T)


def _layer_body(t, n_top, last, x_ref, ct_ref, wq_ref, bq_ref, wk_ref, bk_ref,
                wv_ref, bv_ref, wo_ref, bo_ref, w0_ref, w1_ref, bc_ref,
                g_ref, bb_ref, o_ref):
    f32 = jnp.float32
    x = x_ref[0]
    q = jnp.dot(x, wq_ref[...], preferred_element_type=f32) + bq_ref[...]
    k = jnp.dot(x, wk_ref[...], preferred_element_type=f32) + bk_ref[...]
    v = jnp.dot(x, wv_ref[...], preferred_element_type=f32) + bv_ref[...]

    chunk = min(t, 512)
    m_rows = []
    for h in range(_H):
        sl = slice(_KD * h, _KD * (h + 1))
        qh, kh = q[:, sl], k[:, sl]
        pieces = []
        for c0 in range(0, t, chunk):
            qc = qh[c0:c0 + chunk]
            qkT = jax.lax.dot_general(kh, qc, (((1,), (1,)), ((), ())),
                                      preferred_element_type=f32)
            cf = ct_ref[:, c0:c0 + chunk].astype(f32)
            mx = jnp.max(jnp.where(cf > 0, qkT, -jnp.inf), axis=0,
                         keepdims=True)
            sm = jnp.sum(qkT * cf, axis=0, keepdims=True)
            pieces.append(mx - sm * f32(1.0 / t))
        m_rows.append(pieces[0] if len(pieces) == 1
                      else jnp.concatenate(pieces, axis=1))
    m = jnp.concatenate(m_rows, axis=0)

    iota = jax.lax.broadcasted_iota(jnp.int32, (_H, t), 1)
    ohs = []
    for _ in range(n_top):
        cur = jnp.max(m, axis=1, keepdims=True)
        first = jnp.min(jnp.where(m >= cur, iota, t), axis=1, keepdims=True)
        oh = iota == first
        ohs.append(oh.astype(f32))
        m = jnp.where(oh, -jnp.inf, m)

    acc = None
    for h in range(_H):
        sl = slice(_KD * h, _KD * (h + 1))
        qh, kh, vh = q[:, sl], k[:, sl], v[:, sl]
        onehot = jnp.concatenate([ohs[u][h:h + 1, :] for u in range(n_top)],
                                 axis=0).astype(f32)
        qred = jax.lax.dot_general(onehot, qh, (((1,), (0,)), ((), ())),
                                   preferred_element_type=f32)
        sc = jax.lax.dot_general(qred, kh, (((1,), (1,)), ((), ())),
                                 preferred_element_type=f32)
        sc = sc * f32(1.0 / math.sqrt(_KD))
        sc = sc - jnp.max(sc, axis=1, keepdims=True)
        e = jnp.exp(sc)
        attn = e / jnp.sum(e, axis=1, keepdims=True)
        upd = jax.lax.dot_general(attn, vh, (((1,), (0,)), ((), ())),
                                  preferred_element_type=f32)
        scat = jax.lax.dot_general(onehot, upd, (((0,), (0,)), ((), ())),
                                   preferred_element_type=f32)
        selc = jax.lax.dot_general(onehot, jnp.ones((n_top, 1), f32),
                                   (((0,), (0,)), ((), ())),
                                   preferred_element_type=f32)
        vmean = jnp.mean(vh, axis=0, keepdims=True)
        ctx = scat + (1.0 - selc) * vmean
        part = jnp.dot(ctx, wo_ref[sl, :], preferred_element_type=f32)
        acc = part if acc is None else acc + part
    attn_out = acc + bo_ref[...]

    a0 = jnp.dot(attn_out, w0_ref[...], preferred_element_type=f32)
    a1 = jnp.dot(attn_out, w1_ref[...], preferred_element_type=f32)
    y = jnp.concatenate([jnp.zeros((1, _D), f32), a0[:t - 1]], axis=0) \
        + a1 + bc_ref[...]
    y = jnp.where(y > 0, y, jnp.exp(y) - 1.0)
    pooled = jnp.max(y.reshape(t // 2, 2, _D), axis=1)

    if not last:
        mu = jnp.mean(pooled, axis=1, keepdims=True)
        d = pooled - mu
        var = jnp.mean(d * d, axis=1, keepdims=True)
        pooled = g_ref[...] * (d / jnp.sqrt(var + 1e-3)) + bb_ref[...]

    o_ref[0] = pooled


def _layer_call(x4, pa, pc, ln, layer, t):
    n_top = min(int(_FACTOR * math.ceil(math.log(t))), t)
    ct = jnp.asarray(_sample_count_t(layer, t, n_top))
    last = ln is None
    g = ln['g'] if not last else jnp.ones((_D,), jnp.float32)
    bb = ln['b'] if not last else jnp.zeros((_D,), jnp.float32)
    hk = _H * _KD

    full = lambda *shape: pl.BlockSpec(shape, lambda i: (0,) * len(shape))
    out = pl.pallas_call(
        functools.partial(_layer_body, t, n_top, last),
        grid=(4,),
        in_specs=[
            pl.BlockSpec((1, t, _D), lambda i: (i, 0, 0)),
            full(t, t),
            full(_D, hk), full(1, hk),
            full(_D, hk), full(1, hk),
            full(_D, hk), full(1, hk),
            full(hk, _D), full(1, _D),
            full(_D, _D), full(_D, _D), full(1, _D),
            full(1, _D), full(1, _D),
        ],
        out_specs=pl.BlockSpec((1, t // 2, _D), lambda i: (i, 0, 0)),
        out_shape=jax.ShapeDtypeStruct((4, t // 2, _D), jnp.float32),
        compiler_params=pltpu.CompilerParams(
            dimension_semantics=("arbitrary",)),
    )(x4, ct,
      pa['Wq'], pa['bq'].reshape(1, hk),
      pa['Wk'], pa['bk'].reshape(1, hk),
      pa['Wv'], pa['bv'].reshape(1, hk),
      pa['Wo'], pa['bo'].reshape(1, _D),
      pc['W'][0, 0], pc['W'][0, 1], pc['b'].reshape(1, _D),
      g.reshape(1, _D), bb.reshape(1, _D))
    return out


def kernel(tensor, params):
    b, v, t, d = tensor.shape
    x = tensor.reshape(b * v, t, d)
    for i in range(_LAYERS):
        ln = params['ln_%d' % i] if i < _LAYERS - 1 else None
        x = _layer_call(x, params['attn_%d' % i], params['conv_%d' % i],
                        ln, i, x.shape[1])
    return x.reshape(b, v, x.shape[1], d)


# speedup vs baseline: 72.0897x; 1.0596x over previous
import functools
import math

import numpy as np
import jax
import jax.numpy as jnp
from jax.experimental import pallas as pl
from jax.experimental.pallas import tpu as pltpu

_H = 4
_KD = 16
_VD = 16
_FACTOR = 5
_LAYERS = 3
_DCONV = 2
_D = 128


def _rotl(x, r):
    return ((x << np.uint32(r)) | (x >> np.uint32(32 - r))).astype(np.uint32)


def _threefry2x32(k0, k1, x0, x1):
    with np.errstate(over='ignore'):
        ks0, ks1 = np.uint32(k0), np.uint32(k1)
        ks2 = np.uint32(ks0 ^ ks1 ^ np.uint32(0x1BD11BDA))
        ks = [ks0, ks1, ks2]
        x0 = (x0 + ks0).astype(np.uint32)
        x1 = (x1 + ks1).astype(np.uint32)
        rot = [[13, 15, 26, 6], [17, 29, 16, 24]]
        for i in range(5):
            for r in rot[i % 2]:
                x0 = (x0 + x1).astype(np.uint32)
                x1 = _rotl(x1, r) ^ x0
            x0 = (x0 + ks[(i + 1) % 3]).astype(np.uint32)
            x1 = (x1 + ks[(i + 2) % 3] + np.uint32(i + 1)).astype(np.uint32)
    return x0, x1


def _np_bits(k0, k1, n):
    i = np.arange(n, dtype=np.uint64)
    hi = (i >> np.uint64(32)).astype(np.uint32)
    lo = (i & np.uint64(0xFFFFFFFF)).astype(np.uint32)
    y0, y1 = _threefry2x32(k0, k1, hi, lo)
    return y0 ^ y1


def _np_randint(k0, k1, t, u):
    hi0, hi1 = _threefry2x32(k0, k1, np.uint32(0), np.uint32(0))
    lo0, lo1 = _threefry2x32(k0, k1, np.uint32(0), np.uint32(1))
    y = _np_bits(hi0, hi1, t * u)
    z = _np_bits(lo0, lo1, t * u)
    span = np.uint32(t)
    mult = np.uint32((int(65536 % span) ** 2) % int(span))
    val = ((y % span) * mult + (z % span)) % span
    return val.reshape(t, u).astype(np.int32)


@functools.lru_cache(maxsize=None)
def _sample_consts(layer: int, t: int, u_part: int):
    import ml_dtypes
    k0, k1 = _threefry2x32(np.uint32(0), np.uint32(42),
                           np.uint32(0), np.uint32(layer))
    idx = _np_randint(k0, k1, t, u_part)
    c = np.zeros((t, t), np.float32)
    np.add.at(c, (np.repeat(np.arange(t), u_part), idx.ravel()), 1.0)
    bias_t = np.where(c.T > 0, np.float32(0), np.float32(-np.inf))
    bf16 = ml_dtypes.bfloat16
    return (np.ascontiguousarray(c.astype(bf16)),
            np.ascontiguousarray(bias_t.astype(bf16)))


@functools.lru_cache(maxsize=None)
def _head_consts(n_top: int):
    bm = np.zeros((_H * n_top, _H * _KD), np.float32)
    for h in range(_H):
        bm[h * n_top:(h + 1) * n_top, h * _KD:(h + 1) * _KD] = 1.0
    colseg = np.zeros((_H * _KD, _H), np.float32)
    for h in range(_H):
        colseg[h * _KD:(h + 1) * _KD, h] = 1.0
    return bm, colseg


def _layer_body(t, n_top, last, x_ref, c_ref, bias_ref, bm_ref, cs_ref,
                wq_ref, bq_ref, wk_ref, bk_ref,
                wv_ref, bv_ref, wo_ref, bo_ref, w0_ref, w1_ref, bc_ref,
                g_ref, bb_ref, o_ref):
    f32 = jnp.float32
    x = x_ref[0]
    q = jnp.dot(x, wq_ref[...], preferred_element_type=f32) + bq_ref[...]
    k = jnp.dot(x, wk_ref[...], preferred_element_type=f32) + bk_ref[...]
    v = jnp.dot(x, wv_ref[...], preferred_element_type=f32) + bv_ref[...]

    ksum = jnp.dot(c_ref[...], k.astype(jnp.bfloat16),
                   preferred_element_type=f32)
    smT = jax.lax.dot_general(cs_ref[...], q * ksum, (((0,), (1,)), ((), ())),
                              preferred_element_type=f32)
    chunk = min(t, 512)
    m_rows = []
    for h in range(_H):
        sl = slice(_KD * h, _KD * (h + 1))
        qh, kh = q[:, sl], k[:, sl]
        pieces = []
        for c0 in range(0, t, chunk):
            qc = qh[c0:c0 + chunk]
            qkT = jax.lax.dot_general(kh, qc, (((1,), (1,)), ((), ())),
                                      preferred_element_type=f32)
            bias = bias_ref[:, c0:c0 + chunk].astype(f32)
            pieces.append(jnp.max(qkT + bias, axis=0, keepdims=True))
        m_rows.append(pieces[0] if len(pieces) == 1
                      else jnp.concatenate(pieces, axis=1))
    m = jnp.concatenate(m_rows, axis=0) - smT * f32(1.0 / t)

    iota = jax.lax.broadcasted_iota(jnp.int32, (_H, t), 1)
    ohs = []
    for _ in range(n_top):
        cur = jnp.max(m, axis=1, keepdims=True)
        first = jnp.min(jnp.where(m >= cur, iota, t), axis=1, keepdims=True)
        oh = iota == first
        ohs.append(oh.astype(f32))
        m = jnp.where(oh, -jnp.inf, m)

    bm = bm_ref[...]
    onehot = jnp.concatenate(
        [ohs[u][h:h + 1, :] for h in range(_H) for u in range(n_top)],
        axis=0)
    qred = jnp.dot(onehot, q, preferred_element_type=f32) * bm
    sc = jax.lax.dot_general(qred, k, (((1,), (1,)), ((), ())),
                             preferred_element_type=f32)
    sc = sc * f32(1.0 / math.sqrt(_KD))
    sc = sc - jnp.max(sc, axis=1, keepdims=True)
    e = jnp.exp(sc)
    attn = e / jnp.sum(e, axis=1, keepdims=True)
    upd = jnp.dot(attn, v, preferred_element_type=f32) * bm
    scat = jax.lax.dot_general(onehot, upd, (((0,), (0,)), ((), ())),
                               preferred_element_type=f32)
    sel64 = jax.lax.dot_general(onehot, bm, (((0,), (0,)), ((), ())),
                                preferred_element_type=f32)
    vmean = jnp.mean(v, axis=0, keepdims=True)
    ctx = scat + (1.0 - sel64) * vmean
    attn_out = jnp.dot(ctx, wo_ref[...], preferred_element_type=f32) \
        + bo_ref[...]

    a0 = jnp.dot(attn_out, w0_ref[...], preferred_element_type=f32)
    a1 = jnp.dot(attn_out, w1_ref[...], preferred_element_type=f32)
    y = jnp.concatenate([jnp.zeros((1, _D), f32), a0[:t - 1]], axis=0) \
        + a1 + bc_ref[...]
    y = jnp.where(y > 0, y, jnp.exp(y) - 1.0)
    pooled = jnp.max(y.reshape(t // 2, 2, _D), axis=1)

    if not last:
        mu = jnp.mean(pooled, axis=1, keepdims=True)
        d = pooled - mu
        var = jnp.mean(d * d, axis=1, keepdims=True)
        pooled = g_ref[...] * (d / jnp.sqrt(var + 1e-3)) + bb_ref[...]

    o_ref[0] = pooled


def _layer_call(x4, pa, pc, ln, layer, t):
    n_top = min(int(_FACTOR * math.ceil(math.log(t))), t)
    c_bf, bias_bf = (jnp.asarray(a) for a in _sample_consts(layer, t, n_top))
    bm, colseg = (jnp.asarray(a) for a in _head_consts(n_top))
    last = ln is None
    g = ln['g'] if not last else jnp.ones((_D,), jnp.float32)
    bb = ln['b'] if not last else jnp.zeros((_D,), jnp.float32)
    hk = _H * _KD

    full = lambda *shape: pl.BlockSpec(shape, lambda i: (0,) * len(shape))
    out = pl.pallas_call(
        functools.partial(_layer_body, t, n_top, last),
        grid=(4,),
        in_specs=[
            pl.BlockSpec((1, t, _D), lambda i: (i, 0, 0)),
            full(t, t), full(t, t),
            full(_H * n_top, hk), full(hk, _H),
            full(_D, hk), full(1, hk),
            full(_D, hk), full(1, hk),
            full(_D, hk), full(1, hk),
            full(hk, _D), full(1, _D),
            full(_D, _D), full(_D, _D), full(1, _D),
            full(1, _D), full(1, _D),
        ],
        out_specs=pl.BlockSpec((1, t // 2, _D), lambda i: (i, 0, 0)),
        out_shape=jax.ShapeDtypeStruct((4, t // 2, _D), jnp.float32),
        compiler_params=pltpu.CompilerParams(
            dimension_semantics=("arbitrary",)),
    )(x4, c_bf, bias_bf, bm, colseg,
      pa['Wq'], pa['bq'].reshape(1, hk),
      pa['Wk'], pa['bk'].reshape(1, hk),
      pa['Wv'], pa['bv'].reshape(1, hk),
      pa['Wo'], pa['bo'].reshape(1, _D),
      pc['W'][0, 0], pc['W'][0, 1], pc['b'].reshape(1, _D),
      g.reshape(1, _D), bb.reshape(1, _D))
    return out


def kernel(tensor, params):
    b, v, t, d = tensor.shape
    x = tensor.reshape(b * v, t, d)
    for i in range(_LAYERS):
        ln = params['ln_%d' % i] if i < _LAYERS - 1 else None
        x = _layer_call(x, params['attn_%d' % i], params['conv_%d' % i],
                        ln, i, x.shape[1])
    return x.reshape(b, v, x.shape[1], d)


# speedup vs baseline: 72.1932x; 1.0014x over previous
import functools
import math

import numpy as np
import jax
import jax.numpy as jnp
from jax.experimental import pallas as pl
from jax.experimental.pallas import tpu as pltpu

_H = 4
_KD = 16
_VD = 16
_FACTOR = 5
_LAYERS = 3
_DCONV = 2
_D = 128


def _rotl(x, r):
    return ((x << np.uint32(r)) | (x >> np.uint32(32 - r))).astype(np.uint32)


def _threefry2x32(k0, k1, x0, x1):
    with np.errstate(over='ignore'):
        ks0, ks1 = np.uint32(k0), np.uint32(k1)
        ks2 = np.uint32(ks0 ^ ks1 ^ np.uint32(0x1BD11BDA))
        ks = [ks0, ks1, ks2]
        x0 = (x0 + ks0).astype(np.uint32)
        x1 = (x1 + ks1).astype(np.uint32)
        rot = [[13, 15, 26, 6], [17, 29, 16, 24]]
        for i in range(5):
            for r in rot[i % 2]:
                x0 = (x0 + x1).astype(np.uint32)
                x1 = _rotl(x1, r) ^ x0
            x0 = (x0 + ks[(i + 1) % 3]).astype(np.uint32)
            x1 = (x1 + ks[(i + 2) % 3] + np.uint32(i + 1)).astype(np.uint32)
    return x0, x1


def _np_bits(k0, k1, n):
    i = np.arange(n, dtype=np.uint64)
    hi = (i >> np.uint64(32)).astype(np.uint32)
    lo = (i & np.uint64(0xFFFFFFFF)).astype(np.uint32)
    y0, y1 = _threefry2x32(k0, k1, hi, lo)
    return y0 ^ y1


def _np_randint(k0, k1, t, u):
    hi0, hi1 = _threefry2x32(k0, k1, np.uint32(0), np.uint32(0))
    lo0, lo1 = _threefry2x32(k0, k1, np.uint32(0), np.uint32(1))
    y = _np_bits(hi0, hi1, t * u)
    z = _np_bits(lo0, lo1, t * u)
    span = np.uint32(t)
    mult = np.uint32((int(65536 % span) ** 2) % int(span))
    val = ((y % span) * mult + (z % span)) % span
    return val.reshape(t, u).astype(np.int32)


@functools.lru_cache(maxsize=None)
def _sample_consts(layer: int, t: int, u_part: int):
    import ml_dtypes
    k0, k1 = _threefry2x32(np.uint32(0), np.uint32(42),
                           np.uint32(0), np.uint32(layer))
    idx = _np_randint(k0, k1, t, u_part)
    c = np.zeros((t, t), np.float32)
    np.add.at(c, (np.repeat(np.arange(t), u_part), idx.ravel()), 1.0)
    bias_t = np.where(c.T > 0, np.float32(0), np.float32(-np.inf))
    bf16 = ml_dtypes.bfloat16
    return (np.ascontiguousarray(c.astype(bf16)),
            np.ascontiguousarray(bias_t.astype(bf16)))


@functools.lru_cache(maxsize=None)
def _head_consts(n_top: int):
    bm = np.zeros((_H * n_top, _H * _KD), np.float32)
    for h in range(_H):
        bm[h * n_top:(h + 1) * n_top, h * _KD:(h + 1) * _KD] = 1.0
    colseg = np.zeros((_H * _KD, _H), np.float32)
    for h in range(_H):
        colseg[h * _KD:(h + 1) * _KD, h] = 1.0
    return bm, colseg


def _layer_compute(t, n_top, last, x, c_ref, bias_ref, bm_ref, cs_ref,
                   wq_ref, bq_ref, wk_ref, bk_ref,
                   wv_ref, bv_ref, wo_ref, bo_ref, w0_ref, w1_ref, bc_ref,
                   g_ref, bb_ref):
    f32 = jnp.float32
    q = jnp.dot(x, wq_ref[...], preferred_element_type=f32) + bq_ref[...]
    k = jnp.dot(x, wk_ref[...], preferred_element_type=f32) + bk_ref[...]
    v = jnp.dot(x, wv_ref[...], preferred_element_type=f32) + bv_ref[...]

    ksum = jnp.dot(c_ref[...], k.astype(jnp.bfloat16),
                   preferred_element_type=f32)
    smT = jax.lax.dot_general(cs_ref[...], q * ksum, (((0,), (1,)), ((), ())),
                              preferred_element_type=f32)
    chunk = min(t, 512)
    m_rows = []
    for h in range(_H):
        sl = slice(_KD * h, _KD * (h + 1))
        qh, kh = q[:, sl], k[:, sl]
        pieces = []
        for c0 in range(0, t, chunk):
            qc = qh[c0:c0 + chunk]
            qkT = jax.lax.dot_general(kh, qc, (((1,), (1,)), ((), ())),
                                      preferred_element_type=f32)
            bias = bias_ref[:, c0:c0 + chunk].astype(f32)
            pieces.append(jnp.max(qkT + bias, axis=0, keepdims=True))
        m_rows.append(pieces[0] if len(pieces) == 1
                      else jnp.concatenate(pieces, axis=1))
    m = jnp.concatenate(m_rows, axis=0) - smT * f32(1.0 / t)

    iota = jax.lax.broadcasted_iota(jnp.int32, (_H, t), 1)
    ohs = []
    for _ in range(n_top):
        cur = jnp.max(m, axis=1, keepdims=True)
        first = jnp.min(jnp.where(m >= cur, iota, t), axis=1, keepdims=True)
        oh = iota == first
        ohs.append(oh.astype(f32))
        m = jnp.where(oh, -jnp.inf, m)

    bm = bm_ref[...]
    onehot = jnp.concatenate(
        [ohs[u][h:h + 1, :] for h in range(_H) for u in range(n_top)],
        axis=0)
    qred = jnp.dot(onehot, q, preferred_element_type=f32) * bm
    sc = jax.lax.dot_general(qred, k, (((1,), (1,)), ((), ())),
                             preferred_element_type=f32)
    sc = sc * f32(1.0 / math.sqrt(_KD))
    sc = sc - jnp.max(sc, axis=1, keepdims=True)
    e = jnp.exp(sc)
    attn = e / jnp.sum(e, axis=1, keepdims=True)
    upd = jnp.dot(attn, v, preferred_element_type=f32) * bm
    scat = jax.lax.dot_general(onehot, upd, (((0,), (0,)), ((), ())),
                               preferred_element_type=f32)
    sel64 = jax.lax.dot_general(onehot, bm, (((0,), (0,)), ((), ())),
                                preferred_element_type=f32)
    vmean = jnp.mean(v, axis=0, keepdims=True)
    ctx = scat + (1.0 - sel64) * vmean
    attn_out = jnp.dot(ctx, wo_ref[...], preferred_element_type=f32) \
        + bo_ref[...]

    a0 = jnp.dot(attn_out, w0_ref[...], preferred_element_type=f32)
    a1 = jnp.dot(attn_out, w1_ref[...], preferred_element_type=f32)
    y = jnp.concatenate([jnp.zeros((1, _D), f32), a0[:t - 1]], axis=0) \
        + a1 + bc_ref[...]
    y = jnp.where(y > 0, y, jnp.exp(y) - 1.0)
    pooled = jnp.max(y.reshape(t // 2, 2, _D), axis=1)

    if not last:
        mu = jnp.mean(pooled, axis=1, keepdims=True)
        d = pooled - mu
        var = jnp.mean(d * d, axis=1, keepdims=True)
        pooled = g_ref[...] * (d / jnp.sqrt(var + 1e-3)) + bb_ref[...]

    return pooled


_NPER = 17


def _fused_body(cfgs, *refs):
    x = refs[0][0]
    for i, (t, n_top, last) in enumerate(cfgs):
        lr = refs[1 + i * _NPER:1 + (i + 1) * _NPER]
        x = _layer_compute(t, n_top, last, x, *lr)
    refs[-1][0] = x


def kernel(tensor, params):
    b, v, t0, d = tensor.shape
    x4 = tensor.reshape(b * v, t0, d)
    hk = _H * _KD
    full = lambda *shape: pl.BlockSpec(shape, lambda i: (0,) * len(shape))

    cfgs = []
    in_specs = [pl.BlockSpec((1, t0, _D), lambda i: (i, 0, 0))]
    inputs = [x4]
    t = t0
    for i in range(_LAYERS):
        n_top = min(int(_FACTOR * math.ceil(math.log(t))), t)
        last = i == _LAYERS - 1
        cfgs.append((t, n_top, last))
        c_bf, bias_bf = (jnp.asarray(a)
                         for a in _sample_consts(i, t, n_top))
        bm, colseg = (jnp.asarray(a) for a in _head_consts(n_top))
        ln = params.get('ln_%d' % i)
        g = ln['g'] if not last else jnp.ones((_D,), jnp.float32)
        bb = ln['b'] if not last else jnp.zeros((_D,), jnp.float32)
        pa, pc = params['attn_%d' % i], params['conv_%d' % i]
        inputs += [c_bf, bias_bf, bm, colseg,
                   pa['Wq'], pa['bq'].reshape(1, hk),
                   pa['Wk'], pa['bk'].reshape(1, hk),
                   pa['Wv'], pa['bv'].reshape(1, hk),
                   pa['Wo'], pa['bo'].reshape(1, _D),
                   pc['W'][0, 0], pc['W'][0, 1], pc['b'].reshape(1, _D),
                   g.reshape(1, _D), bb.reshape(1, _D)]
        in_specs += [
            full(t, t), full(t, t),
            full(_H * n_top, hk), full(hk, _H),
            full(_D, hk), full(1, hk),
            full(_D, hk), full(1, hk),
            full(_D, hk), full(1, hk),
            full(hk, _D), full(1, _D),
            full(_D, _D), full(_D, _D), full(1, _D),
            full(1, _D), full(1, _D),
        ]
        t //= 2

    out = pl.pallas_call(
        functools.partial(_fused_body, tuple(cfgs)),
        grid=(4,),
        in_specs=in_specs,
        out_specs=pl.BlockSpec((1, t, _D), lambda i: (i, 0, 0)),
        out_shape=jax.ShapeDtypeStruct((4, t, _D), jnp.float32),
        compiler_params=pltpu.CompilerParams(
            dimension_semantics=("arbitrary",)),
    )(*inputs)
    return out.reshape(b, v, t, d)


# speedup vs baseline: 95.3545x; 1.3208x over previous
import functools
import math

import numpy as np
import jax
import jax.numpy as jnp
from jax.experimental import pallas as pl
from jax.experimental.pallas import tpu as pltpu

_H = 4
_KD = 16
_VD = 16
_FACTOR = 5
_LAYERS = 3
_DCONV = 2
_D = 128


def _rotl(x, r):
    return ((x << np.uint32(r)) | (x >> np.uint32(32 - r))).astype(np.uint32)


def _threefry2x32(k0, k1, x0, x1):
    with np.errstate(over='ignore'):
        ks0, ks1 = np.uint32(k0), np.uint32(k1)
        ks2 = np.uint32(ks0 ^ ks1 ^ np.uint32(0x1BD11BDA))
        ks = [ks0, ks1, ks2]
        x0 = (x0 + ks0).astype(np.uint32)
        x1 = (x1 + ks1).astype(np.uint32)
        rot = [[13, 15, 26, 6], [17, 29, 16, 24]]
        for i in range(5):
            for r in rot[i % 2]:
                x0 = (x0 + x1).astype(np.uint32)
                x1 = _rotl(x1, r) ^ x0
            x0 = (x0 + ks[(i + 1) % 3]).astype(np.uint32)
            x1 = (x1 + ks[(i + 2) % 3] + np.uint32(i + 1)).astype(np.uint32)
    return x0, x1


def _np_bits(k0, k1, n):
    i = np.arange(n, dtype=np.uint64)
    hi = (i >> np.uint64(32)).astype(np.uint32)
    lo = (i & np.uint64(0xFFFFFFFF)).astype(np.uint32)
    y0, y1 = _threefry2x32(k0, k1, hi, lo)
    return y0 ^ y1


def _np_randint(k0, k1, t, u):
    hi0, hi1 = _threefry2x32(k0, k1, np.uint32(0), np.uint32(0))
    lo0, lo1 = _threefry2x32(k0, k1, np.uint32(0), np.uint32(1))
    y = _np_bits(hi0, hi1, t * u)
    z = _np_bits(lo0, lo1, t * u)
    span = np.uint32(t)
    mult = np.uint32((int(65536 % span) ** 2) % int(span))
    val = ((y % span) * mult + (z % span)) % span
    return val.reshape(t, u).astype(np.int32)


@functools.lru_cache(maxsize=None)
def _sample_consts(layer: int, t: int, u_part: int):
    import ml_dtypes
    k0, k1 = _threefry2x32(np.uint32(0), np.uint32(42),
                           np.uint32(0), np.uint32(layer))
    idx = _np_randint(k0, k1, t, u_part)
    c = np.zeros((t, t), np.float32)
    np.add.at(c, (np.repeat(np.arange(t), u_part), idx.ravel()), 1.0)
    bias_t = np.where(c.T > 0, np.float32(0), np.float32(-np.inf))
    bf16 = ml_dtypes.bfloat16
    return (np.ascontiguousarray(c.astype(bf16)),
            np.ascontiguousarray(bias_t.astype(bf16)))


@functools.lru_cache(maxsize=None)
def _head_consts(n_top: int):
    bm = np.zeros((_H * n_top, _H * _KD), np.float32)
    for h in range(_H):
        bm[h * n_top:(h + 1) * n_top, h * _KD:(h + 1) * _KD] = 1.0
    colseg = np.zeros((_H * _KD, _H), np.float32)
    for h in range(_H):
        colseg[h * _KD:(h + 1) * _KD, h] = 1.0
    return bm, colseg


def _layer_body(t, n_top, last, nbv, x_ref, c_ref, bias_ref, bm_ref, cs_ref,
                wq_ref, bq_ref, wk_ref, bk_ref,
                wv_ref, bv_ref, wo_ref, bo_ref, w0_ref, w1_ref, bc_ref,
                g_ref, bb_ref, o_ref):
    f32 = jnp.float32
    qs, ks, vs, smTs = [], [], [], []
    cbf = c_ref[...]
    cseg = cs_ref[...]
    for bv in range(nbv):
        x = x_ref[bv]
        q = jnp.dot(x, wq_ref[...], preferred_element_type=f32) + bq_ref[...]
        k = jnp.dot(x, wk_ref[...], preferred_element_type=f32) + bk_ref[...]
        v = jnp.dot(x, wv_ref[...], preferred_element_type=f32) + bv_ref[...]
        qs.append(q)
        ks.append(k)
        vs.append(v)
        ksum = jnp.dot(cbf, k.astype(jnp.bfloat16),
                       preferred_element_type=f32)
        smTs.append(jax.lax.dot_general(cseg, q * ksum,
                                        (((0,), (1,)), ((), ())),
                                        preferred_element_type=f32))

    chunk = 256 if t >= 2048 else min(t, 512)
    pieces = [[[] for _ in range(_H)] for _ in range(nbv)]
    for c0 in range(0, t, chunk):
        bias = bias_ref[:, c0:c0 + chunk].astype(f32)
        for bv in range(nbv):
            for h in range(_H):
                sl = slice(_KD * h, _KD * (h + 1))
                qc = qs[bv][c0:c0 + chunk, sl]
                qkT = jax.lax.dot_general(ks[bv][:, sl], qc,
                                          (((1,), (1,)), ((), ())),
                                          preferred_element_type=f32)
                pieces[bv][h].append(
                    jnp.max(qkT + bias, axis=0, keepdims=True))
    m_rows = [pieces[bv][h][0] if len(pieces[bv][h]) == 1
              else jnp.concatenate(pieces[bv][h], axis=1)
              for bv in range(nbv) for h in range(_H)]
    sm_all = jnp.concatenate(smTs, axis=0)
    m = jnp.concatenate(m_rows, axis=0) - sm_all * f32(1.0 / t)

    iota = jax.lax.broadcasted_iota(jnp.int32, (nbv * _H, t), 1)
    ohs = []
    for _ in range(n_top):
        cur = jnp.max(m, axis=1, keepdims=True)
        first = jnp.min(jnp.where(m >= cur, iota, t), axis=1, keepdims=True)
        oh = iota == first
        ohs.append(oh.astype(f32))
        m = jnp.where(oh, -jnp.inf, m)

    bm = bm_ref[...]
    for bv in range(nbv):
        q, k, v = qs[bv], ks[bv], vs[bv]
        onehot = jnp.concatenate(
            [ohs[u][bv * _H + h:bv * _H + h + 1, :]
             for h in range(_H) for u in range(n_top)],
            axis=0)
        qred = jnp.dot(onehot, q, preferred_element_type=f32) * bm
        sc = jax.lax.dot_general(qred, k, (((1,), (1,)), ((), ())),
                                 preferred_element_type=f32)
        sc = sc * f32(1.0 / math.sqrt(_KD))
        sc = sc - jnp.max(sc, axis=1, keepdims=True)
        e = jnp.exp(sc)
        attn = e / jnp.sum(e, axis=1, keepdims=True)
        upd = jnp.dot(attn, v, preferred_element_type=f32) * bm
        scat = jax.lax.dot_general(onehot, upd, (((0,), (0,)), ((), ())),
                                   preferred_element_type=f32)
        sel64 = jax.lax.dot_general(onehot, bm, (((0,), (0,)), ((), ())),
                                    preferred_element_type=f32)
        vmean = jnp.mean(v, axis=0, keepdims=True)
        ctx = scat + (1.0 - sel64) * vmean
        attn_out = jnp.dot(ctx, wo_ref[...], preferred_element_type=f32) \
            + bo_ref[...]

        a0 = jnp.dot(attn_out, w0_ref[...], preferred_element_type=f32)
        a1 = jnp.dot(attn_out, w1_ref[...], preferred_element_type=f32)
        y = jnp.concatenate([jnp.zeros((1, _D), f32), a0[:t - 1]], axis=0) \
            + a1 + bc_ref[...]
        y = jnp.where(y > 0, y, jnp.exp(y) - 1.0)
        pooled = jnp.max(y.reshape(t // 2, 2, _D), axis=1)

        if not last:
            mu = jnp.mean(pooled, axis=1, keepdims=True)
            d = pooled - mu
            var = jnp.mean(d * d, axis=1, keepdims=True)
            pooled = g_ref[...] * (d / jnp.sqrt(var + 1e-3)) + bb_ref[...]

        o_ref[bv] = pooled


def kernel(tensor, params):
    b, v, t0, d = tensor.shape
    x = tensor.reshape(b * v, t0, d)
    hk = _H * _KD
    t = t0
    for i in range(_LAYERS):
        n_top = min(int(_FACTOR * math.ceil(math.log(t))), t)
        last = i == _LAYERS - 1
        c_bf, bias_bf = (jnp.asarray(a)
                         for a in _sample_consts(i, t, n_top))
        bm, colseg = (jnp.asarray(a) for a in _head_consts(n_top))
        ln = params.get('ln_%d' % i)
        g = ln['g'] if not last else jnp.ones((_D,), jnp.float32)
        bb = ln['b'] if not last else jnp.zeros((_D,), jnp.float32)
        pa, pc = params['attn_%d' % i], params['conv_%d' % i]
        w_args = (pa['Wq'], pa['bq'].reshape(1, hk),
                  pa['Wk'], pa['bk'].reshape(1, hk),
                  pa['Wv'], pa['bv'].reshape(1, hk),
                  pa['Wo'], pa['bo'].reshape(1, _D),
                  pc['W'][0, 0], pc['W'][0, 1], pc['b'].reshape(1, _D),
                  g.reshape(1, _D), bb.reshape(1, _D))
        nsplit = 2 if t >= 2048 else 1
        nbv = (b * v) // nsplit
        outs = []
        for s in range(nsplit):
            outs.append(pl.pallas_call(
                functools.partial(_layer_body, t, n_top, last, nbv),
                out_shape=jax.ShapeDtypeStruct((nbv, t // 2, _D),
                                               jnp.float32),
            )(x[s * nbv:(s + 1) * nbv], c_bf, bias_bf, bm, colseg, *w_args))
        x = outs[0] if nsplit == 1 else jnp.concatenate(outs, axis=0)
        t //= 2
    return x.reshape(b, v, t, d)
